# baseline (speedup 1.0000x reference)
import functools

import jax
import jax.numpy as jnp
from jax import lax
from jax.experimental import pallas as pl
from jax.experimental.pallas import tpu as pltpu
from jax.experimental.pallas import tpu_sc as plsc

_NEG = float(-3.0e38)


def _make_sc_max(batch, seq, d_model):
    info = plsc.get_sparse_core_info()
    nc, ns, lanes = info.num_cores, info.num_subcores, info.num_lanes
    nw = nc * ns
    strips = nw // batch
    fw = d_model // strips
    g = fw // lanes
    nch = 4
    ch = seq // nch
    ru = 8

    mesh = plsc.VectorSubcoreMesh(core_axis_name="c", subcore_axis_name="s")

    @functools.partial(
        pl.kernel, mesh=mesh,
        out_type=jax.ShapeDtypeStruct((batch, d_model), jnp.float32),
        compiler_params=pltpu.CompilerParams(use_tc_tiling_on_sc=False),
        scratch_types=[
            pltpu.VMEM((ch, fw), jnp.float32),
            pltpu.VMEM((ch, fw), jnp.float32),
            pltpu.VMEM((fw,), jnp.float32),
            pltpu.SemaphoreType.DMA,
            pltpu.SemaphoreType.DMA,
        ],
    )
    def sc_max(x_hbm, out_hbm, buf0, buf1, acc_vm, sem0, sem1):
        wid = lax.axis_index("s") * nc + lax.axis_index("c")
        b = wid // strips
        f0 = (wid % strips) * fw

        bufs = (buf0, buf1)
        sems = (sem0, sem1)
        copies = [
            pltpu.make_async_copy(
                x_hbm.at[b, pl.ds(ci * ch, ch), pl.ds(f0, fw)],
                bufs[ci % 2], sems[ci % 2])
            for ci in range(nch)
        ]
        copies[0].start()
        accs = tuple(jnp.full((lanes,), _NEG, jnp.float32) for _ in range(g))
        for ci in range(nch):
            if ci + 1 < nch:
                copies[ci + 1].start()
            copies[ci].wait()
            buf = bufs[ci % 2]

            def step(i, accs, buf=buf):
                out = list(accs)
                for r in range(ru):
                    row = i * ru + r
                    for gi in range(g):
                        out[gi] = jnp.maximum(
                            out[gi], buf[row, pl.ds(gi * lanes, lanes)])
                return tuple(out)

            accs = lax.fori_loop(0, ch // ru, step, accs, unroll=1)
        for gi in range(g):
            acc_vm[pl.ds(gi * lanes, lanes)] = accs[gi]
        pltpu.sync_copy(acc_vm, out_hbm.at[b, pl.ds(f0, fw)])

    return sc_max


def _tail_body(xmax_ref, pk_ref, idx_ref, sim_ref, bkn_ref, rs_ref,
               *, batch, pool, topk):
    xmax = xmax_ref[...]
    pk = pk_ref[...]
    pn = pk * jax.lax.rsqrt(
        jnp.maximum(jnp.sum(pk * pk, axis=1, keepdims=True), 1e-12))
    xn = xmax * jax.lax.rsqrt(
        jnp.maximum(jnp.sum(xmax * xmax, axis=1, keepdims=True), 1e-12))
    sim = jax.lax.dot_general(
        xn, pn, (((1,), (1,)), ((), ())),
        preferred_element_type=jnp.float32)
    iota = jax.lax.broadcasted_iota(jnp.int32, (batch, pool), 1)
    big = jnp.int32(pool)
    neg = jnp.float32(-jnp.inf)
    v1 = jnp.max(sim, axis=1, keepdims=True)
    i1 = jnp.min(jnp.where(sim == v1, iota, big), axis=1, keepdims=True)
    sim2 = jnp.where(iota == i1, neg, sim)
    v2 = jnp.max(sim2, axis=1, keepdims=True)
    i2 = jnp.min(jnp.where(sim2 == v2, iota, big), axis=1, keepdims=True)
    idx = jnp.concatenate([i1, i2], axis=1)
    oh1 = (iota == i1).astype(jnp.float32)
    oh2 = (iota == i2).astype(jnp.float32)
    bkn1 = jax.lax.dot_general(
        oh1, pn, (((1,), (0,)), ((), ())),
        preferred_element_type=jnp.float32)
    bkn2 = jax.lax.dot_general(
        oh2, pn, (((1,), (0,)), ((), ())),
        preferred_element_type=jnp.float32)
    idx_ref[...] = idx
    sim_ref[...] = sim
    bkn_ref[0] = bkn1
    bkn_ref[1] = bkn2
    rs_ref[...] = ((jnp.sum(v1) + jnp.sum(v2)) / batch)[None, None]


def _tc_tail(xmax, prompt_key):
    batch, d_model = xmax.shape
    pool = prompt_key.shape[0]
    topk = 2
    return pl.pallas_call(
        functools.partial(_tail_body, batch=batch, pool=pool, topk=topk),
        out_shape=[
            jax.ShapeDtypeStruct((batch, topk), jnp.int32),
            jax.ShapeDtypeStruct((batch, pool), jnp.float32),
            jax.ShapeDtypeStruct((topk, batch, d_model), jnp.float32),
            jax.ShapeDtypeStruct((1, 1), jnp.float32),
        ],
    )(xmax, prompt_key)


def kernel(x_embed, prompt_key):
    batch, seq, d_model = x_embed.shape
    xmax = _make_sc_max(batch, seq, d_model)(x_embed)
    idx, sim, bkn, rs = _tc_tail(xmax, prompt_key)
    return (idx, sim, bkn.transpose(1, 0, 2), rs.reshape(()))


# speedup vs baseline: 4.5680x; 4.5680x over previous
import functools

import jax
import jax.numpy as jnp
from jax.experimental import pallas as pl
from jax.experimental.pallas import tpu as pltpu


def _body(x1_ref, x2_ref, x3_ref, x4_ref, pk_ref, idx_ref, sim_ref, bkn_ref,
          rs_ref, xmax_ref, *, batch, pool, topk, d_model):
    b = pl.program_id(0)
    xm = jnp.maximum(
        jnp.maximum(jnp.max(x1_ref[0], axis=0), jnp.max(x2_ref[0], axis=0)),
        jnp.maximum(jnp.max(x3_ref[0], axis=0), jnp.max(x4_ref[0], axis=0)))
    xmax_ref[pl.ds(b, 1), :] = xm[None, :]

    @pl.when(b == batch - 1)
    def _tail():
        xmax = xmax_ref[0:batch, :]
        pk = pk_ref[...]
        pn = pk * jax.lax.rsqrt(
            jnp.maximum(jnp.sum(pk * pk, axis=1, keepdims=True), 1e-12))
        xn = xmax * jax.lax.rsqrt(
            jnp.maximum(jnp.sum(xmax * xmax, axis=1, keepdims=True), 1e-12))
        sim = jax.lax.dot_general(
            xn, pn, (((1,), (1,)), ((), ())),
            preferred_element_type=jnp.float32)
        iota = jax.lax.broadcasted_iota(jnp.int32, (batch, pool), 1)
        big = jnp.int32(pool)
        neg = jnp.float32(-jnp.inf)
        v1 = jnp.max(sim, axis=1, keepdims=True)
        i1 = jnp.min(jnp.where(sim == v1, iota, big), axis=1, keepdims=True)
        sim2 = jnp.where(iota == i1, neg, sim)
        v2 = jnp.max(sim2, axis=1, keepdims=True)
        i2 = jnp.min(jnp.where(sim2 == v2, iota, big), axis=1, keepdims=True)
        idx = jnp.concatenate([i1, i2], axis=1)
        oh1 = (iota == i1).astype(jnp.float32)
        oh2 = (iota == i2).astype(jnp.float32)
        bkn1 = jax.lax.dot_general(
            oh1, pn, (((1,), (0,)), ((), ())),
            preferred_element_type=jnp.float32)
        bkn2 = jax.lax.dot_general(
            oh2, pn, (((1,), (0,)), ((), ())),
            preferred_element_type=jnp.float32)
        idx_ref[...] = idx
        sim_ref[...] = sim
        bkn_ref[0] = bkn1
        bkn_ref[1] = bkn2
        rs_ref[...] = ((jnp.sum(v1) + jnp.sum(v2)) / batch)[None, None]


def kernel(x_embed, prompt_key):
    batch, seq, d_model = x_embed.shape
    pool = prompt_key.shape[0]
    topk = 2
    qs = seq // 4

    out = pl.pallas_call(
        functools.partial(_body, batch=batch, pool=pool, topk=topk,
                          d_model=d_model),
        grid=(batch,),
        in_specs=[
            pl.BlockSpec((1, qs, d_model), lambda b: (b, 0, 0)),
            pl.BlockSpec((1, qs, d_model), lambda b: (b, 1, 0)),
            pl.BlockSpec((1, qs, d_model), lambda b: (b, 2, 0)),
            pl.BlockSpec((1, qs, d_model), lambda b: (b, 3, 0)),
            pl.BlockSpec((pool, d_model), lambda b: (0, 0)),
        ],
        out_specs=[
            pl.BlockSpec((batch, topk), lambda b: (0, 0)),
            pl.BlockSpec((batch, pool), lambda b: (0, 0)),
            pl.BlockSpec((topk, batch, d_model), lambda b: (0, 0, 0)),
            pl.BlockSpec((1, 1), lambda b: (0, 0)),
        ],
        out_shape=[
            jax.ShapeDtypeStruct((batch, topk), jnp.int32),
            jax.ShapeDtypeStruct((batch, pool), jnp.float32),
            jax.ShapeDtypeStruct((topk, batch, d_model), jnp.float32),
            jax.ShapeDtypeStruct((1, 1), jnp.float32),
        ],
        scratch_shapes=[pltpu.VMEM((max(batch, 8), d_model), jnp.float32)],
    )(x_embed, x_embed, x_embed, x_embed, prompt_key)

    idx, sim, bkn, rs = out
    return (idx, sim, bkn.transpose(1, 0, 2), rs.reshape(()))


# speedup vs baseline: 4.6546x; 1.0190x over previous
import functools

import jax
import jax.numpy as jnp
from jax.experimental import pallas as pl
from jax.experimental.pallas import tpu as pltpu


def _body(x1_ref, x2_ref, pk_ref, idx_ref, sim_ref, bkn_ref, rs_ref,
          pn_ref, *, batch, pool, topk, d_model):
    b = pl.program_id(0)

    @pl.when(b == 0)
    def _prompt_norm():
        pk = pk_ref[...]
        pn_ref[...] = pk * jax.lax.rsqrt(
            jnp.maximum(jnp.sum(pk * pk, axis=1, keepdims=True), 1e-12))

    xm = jnp.maximum(jnp.max(x1_ref[0], axis=0), jnp.max(x2_ref[0], axis=0))
    ssq = jnp.sum(xm * xm)
    xn = (xm * jax.lax.rsqrt(jnp.maximum(ssq, 1e-12)))[None, :]
    sim_ref[pl.ds(b, 1), :] = jax.lax.dot_general(
        xn, pn_ref[...], (((1,), (1,)), ((), ())),
        preferred_element_type=jnp.float32)

    @pl.when(b == batch - 1)
    def _tail():
        pn = pn_ref[...]
        sim = sim_ref[...]
        iota = jax.lax.broadcasted_iota(jnp.int32, (batch, pool), 1)
        big = jnp.int32(pool)
        neg = jnp.float32(-jnp.inf)
        v1 = jnp.max(sim, axis=1, keepdims=True)
        i1 = jnp.min(jnp.where(sim == v1, iota, big), axis=1, keepdims=True)
        sim2 = jnp.where(iota == i1, neg, sim)
        v2 = jnp.max(sim2, axis=1, keepdims=True)
        i2 = jnp.min(jnp.where(sim2 == v2, iota, big), axis=1, keepdims=True)
        idx_ref[...] = jnp.concatenate([i1, i2], axis=1)
        oh1 = (iota == i1).astype(jnp.float32)
        oh2 = (iota == i2).astype(jnp.float32)
        bkn_ref[0] = jax.lax.dot_general(
            oh1, pn, (((1,), (0,)), ((), ())),
            preferred_element_type=jnp.float32)
        bkn_ref[1] = jax.lax.dot_general(
            oh2, pn, (((1,), (0,)), ((), ())),
            preferred_element_type=jnp.float32)
        rs_ref[...] = ((jnp.sum(v1) + jnp.sum(v2)) / batch)[None, None]


def kernel(x_embed, prompt_key):
    batch, seq, d_model = x_embed.shape
    pool = prompt_key.shape[0]
    topk = 2
    hs = seq // 2

    out = pl.pallas_call(
        functools.partial(_body, batch=batch, pool=pool, topk=topk,
                          d_model=d_model),
        grid=(batch,),
        in_specs=[
            pl.BlockSpec((1, hs, d_model), lambda b: (b, 0, 0)),
            pl.BlockSpec((1, hs, d_model), lambda b: (b, 1, 0)),
            pl.BlockSpec((pool, d_model), lambda b: (0, 0)),
        ],
        out_specs=[
            pl.BlockSpec((batch, topk), lambda b: (0, 0)),
            pl.BlockSpec((batch, pool), lambda b: (0, 0)),
            pl.BlockSpec((topk, batch, d_model), lambda b: (0, 0, 0)),
            pl.BlockSpec((1, 1), lambda b: (0, 0)),
        ],
        out_shape=[
            jax.ShapeDtypeStruct((batch, topk), jnp.int32),
            jax.ShapeDtypeStruct((batch, pool), jnp.float32),
            jax.ShapeDtypeStruct((topk, batch, d_model), jnp.float32),
            jax.ShapeDtypeStruct((1, 1), jnp.float32),
        ],
        scratch_shapes=[pltpu.VMEM((pool, d_model), jnp.float32)],
    )(x_embed, x_embed, prompt_key)

    idx, sim, bkn, rs = out
    return (idx, sim, bkn.transpose(1, 0, 2), rs.reshape(()))
